# baseline (speedup 1.0000x reference)
import functools

import jax
import jax.numpy as jnp
from jax import lax
from jax.experimental import pallas as pl
from jax.experimental.pallas import tpu as pltpu
from jax.experimental.pallas import tpu_sc as plsc

VOCAB = 1000000
EMB = 64
BATCH = 16384

NC = 2
NS = 16
NW = NC * NS
HPW = BATCH // NW
LANES = 16

CHW = 128
CSH = 7
TAIL0 = (VOCAB // 128) * 128
TAILN = VOCAB - TAIL0
LASTC = TAIL0 // CHW - 1
RING = 8

_mesh = plsc.VectorSubcoreMesh(core_axis_name="c", subcore_axis_name="s")
_params = pltpu.CompilerParams(
    needs_layout_passes=False, use_tc_tiling_on_sc=True)


@functools.partial(
    pl.kernel,
    mesh=_mesh,
    compiler_params=_params,
    out_type=(jax.ShapeDtypeStruct((BATCH * EMB,), jnp.float32),
              jax.ShapeDtypeStruct((BATCH * EMB,), jnp.float32)),
    scratch_types=[
        pltpu.VMEM((2, EMB, CHW), jnp.float32),
        pltpu.VMEM((TAILN * EMB,), jnp.float32),
        pltpu.VMEM((RING * EMB,), jnp.float32),
        pltpu.VMEM((HPW + LANES,), jnp.int32),
        pltpu.VMEM((HPW,), jnp.int32),
        pltpu.SemaphoreType.DMA,
        pltpu.SemaphoreType.DMA,
    ],
)
def _gather_sc(us_hbm, uo_hbm, vs_hbm, vo_hbm, u_t_hbm, v_t_hbm,
               u_tail_hbm, v_tail_hbm, su_hbm, sv_hbm,
               chunk_v, tail_v, ring_v, rs_v, bs_v, csem, rsem):
    wid = lax.axis_index("s") * NC + lax.axis_index("c")
    j0 = wid * HPW
    lane = lax.iota(jnp.int32, LANES)

    def chunk_wait():
        pltpu.make_async_copy(
            u_t_hbm.at[:, pl.ds(0, CHW)], chunk_v.at[0], csem).wait()

    def row_wait():
        pltpu.make_async_copy(
            su_hbm.at[pl.ds(0, EMB)], ring_v.at[pl.ds(0, EMB)], rsem).wait()

    def one_table(s_hbm, o_hbm, t_hbm, tail_hbm, stage_hbm):
        pltpu.sync_copy(s_hbm.at[pl.ds(j0, HPW)], rs_v.at[pl.ds(0, HPW)])
        rs_v[pl.ds(HPW, LANES)] = jnp.full((LANES,), VOCAB, jnp.int32)
        pltpu.sync_copy(o_hbm.at[pl.ds(j0, HPW)], bs_v)
        pltpu.sync_copy(tail_hbm, tail_v)

        c0 = jnp.minimum(rs_v[pl.ds(0, LANES)][0] >> CSH, LASTC)
        pltpu.async_copy(t_hbm.at[:, pl.ds(c0 * CHW, CHW)],
                         chunk_v.at[0], csem)
        chunk_wait()

        def group(g, carry):
            cur, pbuf = carry
            rvec = rs_v[pl.ds(g * LANES, LANES)]
            rnvec = rs_v[pl.ds(g * LANES + 1, LANES)]
            bvec = bs_v[pl.ds(g * LANES, LANES)]
            for l in range(LANES):
                r = rvec[l]
                rn = rnvec[l]
                b = bvec[l]
                is_tail = r >= TAIL0
                cs = jnp.where(is_tail, cur, jnp.minimum(r >> CSH, LASTC))
                switch = cs != cur
                pbuf = jnp.where(switch, pbuf ^ 1, pbuf)

                @pl.when(switch)
                def _():
                    chunk_wait()

                cn = jnp.where(rn >= TAIL0, cs, rn >> CSH)

                @pl.when(cn != cs)
                def _(cn=cn, pbuf=pbuf):
                    pltpu.async_copy(t_hbm.at[:, pl.ds(cn * CHW, CHW)],
                                     chunk_v.at[pbuf ^ 1], csem)

                slot = (l % RING) * EMB
                if l >= RING:
                    row_wait()
                else:
                    @pl.when(g > 0)
                    def _():
                        row_wait()

                rel = jnp.clip(r - cs * CHW, 0, CHW - 1)
                p_vec = jnp.full((LANES,), pbuf, jnp.int32)
                rel_vec = jnp.full((LANES,), rel, jnp.int32)
                tbase = jnp.clip(r - TAIL0, 0, TAILN - 1) * EMB
                tail_flag = jnp.full((LANES,), is_tail, jnp.bool_)
                for k in range(EMB // LANES):
                    dk = lane + k * LANES
                    main_vals = plsc.load_gather(
                        chunk_v, [p_vec, dk, rel_vec])
                    tail_vals = plsc.load_gather(tail_v, [tbase + dk])
                    vals = jnp.where(tail_flag, tail_vals, main_vals)
                    ring_v[pl.ds(slot + k * LANES, LANES)] = vals

                pltpu.async_copy(ring_v.at[pl.ds(slot, EMB)],
                                 stage_hbm.at[pl.ds(b * EMB, EMB)], rsem)
                cur = cs
            return (cur, pbuf)

        lax.fori_loop(0, HPW // LANES, group, (c0, jnp.int32(0)))
        for _ in range(RING):
            row_wait()

    one_table(us_hbm, uo_hbm, u_t_hbm, u_tail_hbm, su_hbm)
    one_table(vs_hbm, vo_hbm, v_t_hbm, v_tail_hbm, sv_hbm)


@functools.partial(
    pl.kernel,
    mesh=_mesh,
    compiler_params=_params,
    out_type=jax.ShapeDtypeStruct((BATCH,), jnp.float32),
    scratch_types=[
        pltpu.VMEM((HPW * EMB,), jnp.float32),
        pltpu.VMEM((HPW * EMB,), jnp.float32),
        pltpu.VMEM((HPW,), jnp.float32),
        pltpu.SemaphoreType.DMA,
    ],
)
def _dot_sc(su_hbm, sv_hbm, out_hbm, uf_v, vf_v, out_v, sem):
    wid = lax.axis_index("s") * NC + lax.axis_index("c")
    base = wid * HPW
    cu = pltpu.async_copy(su_hbm.at[pl.ds(base * EMB, HPW * EMB)], uf_v, sem)
    cv = pltpu.async_copy(sv_hbm.at[pl.ds(base * EMB, HPW * EMB)], vf_v, sem)
    cu.wait()
    cv.wait()

    lane = lax.iota(jnp.int32, LANES)

    def blk(i, carry):
        res = jnp.zeros((LANES,), jnp.float32)
        for l in range(LANES):
            roff = (i * LANES + l) * EMB
            acc = jnp.zeros((LANES,), jnp.float32)
            for k in range(EMB // LANES):
                u = uf_v[pl.ds(roff + k * LANES, LANES)]
                v = vf_v[pl.ds(roff + k * LANES, LANES)]
                acc = acc + u * v
            s = jnp.sum(acc)
            res = jnp.where(lane == l, s, res)
        out_v[pl.ds(i * LANES, LANES)] = res
        return carry

    lax.fori_loop(0, HPW // LANES, blk, 0)
    pltpu.sync_copy(out_v, out_hbm.at[pl.ds(base, HPW)])


def kernel(u_idxs, v_idxs, u_weight, v_weight):
    ui = u_idxs.astype(jnp.int32)
    vi = v_idxs.astype(jnp.int32)
    uo = jnp.argsort(ui).astype(jnp.int32)
    vo = jnp.argsort(vi).astype(jnp.int32)
    us = ui[uo]
    vs = vi[vo]
    u_tail = u_weight[TAIL0:, :].reshape(-1)
    v_tail = v_weight[TAIL0:, :].reshape(-1)
    su, sv = _gather_sc(us, uo, vs, vo, u_weight.T, v_weight.T,
                        u_tail, v_tail)
    return _dot_sc(su, sv)


# speedup vs baseline: 1.4280x; 1.4280x over previous
import functools

import jax
import jax.numpy as jnp
from jax import lax
from jax.experimental import pallas as pl
from jax.experimental.pallas import tpu as pltpu
from jax.experimental.pallas import tpu_sc as plsc

VOCAB = 1000000
EMB = 64
BATCH = 16384

NC = 2
NS = 16
NW = NC * NS
HPW = BATCH // NW
LANES = 16

CHW = 512
CSH = 9
TAIL0 = (VOCAB // 128) * 128
TAILN = VOCAB - TAIL0
LASTC = TAIL0 // CHW - 1
RING = 8

_mesh = plsc.VectorSubcoreMesh(core_axis_name="c", subcore_axis_name="s")
_params = pltpu.CompilerParams(
    needs_layout_passes=False, use_tc_tiling_on_sc=True)


@functools.partial(
    pl.kernel,
    mesh=_mesh,
    compiler_params=_params,
    out_type=(jax.ShapeDtypeStruct((BATCH * EMB,), jnp.float32),
              jax.ShapeDtypeStruct((BATCH * EMB,), jnp.float32)),
    scratch_types=[
        pltpu.VMEM((2, EMB, CHW), jnp.float32),
        pltpu.VMEM((TAILN * EMB,), jnp.float32),
        pltpu.VMEM((RING * EMB,), jnp.float32),
        pltpu.VMEM((HPW + LANES,), jnp.int32),
        pltpu.VMEM((HPW,), jnp.int32),
        pltpu.SemaphoreType.DMA,
        pltpu.SemaphoreType.DMA,
    ],
)
def _gather_sc(us_hbm, uo_hbm, vs_hbm, vo_hbm, u_t_hbm, v_t_hbm,
               u_tail_hbm, v_tail_hbm, su_hbm, sv_hbm,
               chunk_v, tail_v, ring_v, rs_v, bs_v, csem, rsem):
    wid = lax.axis_index("s") * NC + lax.axis_index("c")
    j0 = wid * HPW
    lane = lax.iota(jnp.int32, LANES)

    def chunk_wait():
        pltpu.make_async_copy(
            u_t_hbm.at[:, pl.ds(0, CHW)], chunk_v.at[0], csem).wait()

    def row_wait():
        pltpu.make_async_copy(
            su_hbm.at[pl.ds(0, EMB)], ring_v.at[pl.ds(0, EMB)], rsem).wait()

    def one_table(s_hbm, o_hbm, t_hbm, tail_hbm, stage_hbm):
        pltpu.sync_copy(s_hbm.at[pl.ds(j0, HPW)], rs_v.at[pl.ds(0, HPW)])
        rs_v[pl.ds(HPW, LANES)] = jnp.full((LANES,), VOCAB, jnp.int32)
        pltpu.sync_copy(o_hbm.at[pl.ds(j0, HPW)], bs_v)
        pltpu.sync_copy(tail_hbm, tail_v)

        c0 = jnp.minimum(rs_v[pl.ds(0, LANES)][0] >> CSH, LASTC)
        pltpu.async_copy(t_hbm.at[:, pl.ds(c0 * CHW, CHW)],
                         chunk_v.at[0], csem)
        chunk_wait()

        def group(g, carry):
            cur, pbuf = carry
            rvec = rs_v[pl.ds(g * LANES, LANES)]
            rnvec = rs_v[pl.ds(g * LANES + 1, LANES)]
            bvec = bs_v[pl.ds(g * LANES, LANES)]
            for l in range(LANES):
                r = rvec[l]
                rn = rnvec[l]
                b = bvec[l]
                is_tail = r >= TAIL0
                cs = jnp.where(is_tail, cur, jnp.minimum(r >> CSH, LASTC))
                switch = cs != cur
                pbuf = jnp.where(switch, pbuf ^ 1, pbuf)

                @pl.when(switch)
                def _():
                    chunk_wait()

                cn = jnp.where(rn >= TAIL0, cs, rn >> CSH)

                @pl.when(cn != cs)
                def _(cn=cn, pbuf=pbuf):
                    pltpu.async_copy(t_hbm.at[:, pl.ds(cn * CHW, CHW)],
                                     chunk_v.at[pbuf ^ 1], csem)

                slot = (l % RING) * EMB
                if l >= RING:
                    row_wait()
                else:
                    @pl.when(g > 0)
                    def _():
                        row_wait()

                rel = jnp.clip(r - cs * CHW, 0, CHW - 1)
                p_vec = jnp.full((LANES,), pbuf, jnp.int32)
                rel_vec = jnp.full((LANES,), rel, jnp.int32)
                tbase = jnp.clip(r - TAIL0, 0, TAILN - 1) * EMB
                tail_flag = jnp.full((LANES,), is_tail, jnp.bool_)
                for k in range(EMB // LANES):
                    dk = lane + k * LANES
                    main_vals = plsc.load_gather(
                        chunk_v, [p_vec, dk, rel_vec])
                    tail_vals = plsc.load_gather(tail_v, [tbase + dk])
                    vals = jnp.where(tail_flag, tail_vals, main_vals)
                    ring_v[pl.ds(slot + k * LANES, LANES)] = vals

                pltpu.async_copy(ring_v.at[pl.ds(slot, EMB)],
                                 stage_hbm.at[pl.ds(b * EMB, EMB)], rsem)
                cur = cs
            return (cur, pbuf)

        lax.fori_loop(0, HPW // LANES, group, (c0, jnp.int32(0)))
        for _ in range(RING):
            row_wait()

    one_table(us_hbm, uo_hbm, u_t_hbm, u_tail_hbm, su_hbm)
    one_table(vs_hbm, vo_hbm, v_t_hbm, v_tail_hbm, sv_hbm)


@functools.partial(
    pl.kernel,
    mesh=_mesh,
    compiler_params=_params,
    out_type=jax.ShapeDtypeStruct((BATCH,), jnp.float32),
    scratch_types=[
        pltpu.VMEM((HPW * EMB,), jnp.float32),
        pltpu.VMEM((HPW * EMB,), jnp.float32),
        pltpu.VMEM((HPW,), jnp.float32),
        pltpu.SemaphoreType.DMA,
    ],
)
def _dot_sc(su_hbm, sv_hbm, out_hbm, uf_v, vf_v, out_v, sem):
    wid = lax.axis_index("s") * NC + lax.axis_index("c")
    base = wid * HPW
    cu = pltpu.async_copy(su_hbm.at[pl.ds(base * EMB, HPW * EMB)], uf_v, sem)
    cv = pltpu.async_copy(sv_hbm.at[pl.ds(base * EMB, HPW * EMB)], vf_v, sem)
    cu.wait()
    cv.wait()

    lane = lax.iota(jnp.int32, LANES)

    def blk(i, carry):
        res = jnp.zeros((LANES,), jnp.float32)
        for l in range(LANES):
            roff = (i * LANES + l) * EMB
            acc = jnp.zeros((LANES,), jnp.float32)
            for k in range(EMB // LANES):
                u = uf_v[pl.ds(roff + k * LANES, LANES)]
                v = vf_v[pl.ds(roff + k * LANES, LANES)]
                acc = acc + u * v
            s = jnp.sum(acc)
            res = jnp.where(lane == l, s, res)
        out_v[pl.ds(i * LANES, LANES)] = res
        return carry

    lax.fori_loop(0, HPW // LANES, blk, 0)
    pltpu.sync_copy(out_v, out_hbm.at[pl.ds(base, HPW)])


def kernel(u_idxs, v_idxs, u_weight, v_weight):
    ui = u_idxs.astype(jnp.int32)
    vi = v_idxs.astype(jnp.int32)
    uo = jnp.argsort(ui).astype(jnp.int32)
    vo = jnp.argsort(vi).astype(jnp.int32)
    us = ui[uo]
    vs = vi[vo]
    u_tail = u_weight[TAIL0:, :].reshape(-1)
    v_tail = v_weight[TAIL0:, :].reshape(-1)
    su, sv = _gather_sc(us, uo, vs, vo, u_weight.T, v_weight.T,
                        u_tail, v_tail)
    return _dot_sc(su, sv)


# speedup vs baseline: 1.6332x; 1.1437x over previous
import functools

import jax
import jax.numpy as jnp
from jax import lax
from jax.experimental import pallas as pl
from jax.experimental.pallas import tpu as pltpu
from jax.experimental.pallas import tpu_sc as plsc

VOCAB = 1000000
EMB = 64
BATCH = 16384

NC = 2
NS = 16
NW = NC * NS
HPW = BATCH // NW
LANES = 16

CHW = 512
CSH = 9
TAIL0 = (VOCAB // 128) * 128
TAILN = VOCAB - TAIL0
LASTC = TAIL0 // CHW - 1
RING = 8

_mesh = plsc.VectorSubcoreMesh(core_axis_name="c", subcore_axis_name="s")
_params = pltpu.CompilerParams(
    needs_layout_passes=False, use_tc_tiling_on_sc=True)


@functools.partial(
    pl.kernel,
    mesh=_mesh,
    compiler_params=_params,
    out_type=(jax.ShapeDtypeStruct((BATCH * EMB,), jnp.float32),
              jax.ShapeDtypeStruct((BATCH * EMB,), jnp.float32)),
    scratch_types=[
        pltpu.VMEM((2, EMB, CHW), jnp.float32),
        pltpu.VMEM((TAILN * EMB,), jnp.float32),
        pltpu.VMEM((RING * EMB,), jnp.float32),
        pltpu.VMEM((HPW + LANES,), jnp.int32),
        pltpu.VMEM((HPW,), jnp.int32),
        pltpu.SemaphoreType.DMA,
        pltpu.SemaphoreType.DMA,
    ],
)
def _gather_sc(us_hbm, uo_hbm, vs_hbm, vo_hbm, u_t_hbm, v_t_hbm,
               u_tail_hbm, v_tail_hbm, su_hbm, sv_hbm,
               chunk_v, tail_v, ring_v, rs_v, bs_v, csem, rsem):
    wid = lax.axis_index("s") * NC + lax.axis_index("c")
    j0 = wid * HPW
    lane = lax.iota(jnp.int32, LANES)

    def chunk_wait():
        pltpu.make_async_copy(
            u_t_hbm.at[:, pl.ds(0, CHW)], chunk_v.at[0], csem).wait()

    def row_wait():
        pltpu.make_async_copy(
            su_hbm.at[pl.ds(0, EMB)], ring_v.at[pl.ds(0, EMB)], rsem).wait()

    def one_table(s_hbm, o_hbm, t_hbm, tail_hbm, stage_hbm):
        pltpu.sync_copy(s_hbm.at[pl.ds(j0, HPW)], rs_v.at[pl.ds(0, HPW)])
        rs_v[pl.ds(HPW, LANES)] = jnp.full((LANES,), VOCAB, jnp.int32)
        pltpu.sync_copy(o_hbm.at[pl.ds(j0, HPW)], bs_v)
        pltpu.sync_copy(tail_hbm, tail_v)

        c0 = jnp.minimum(rs_v[pl.ds(0, LANES)][0] >> CSH, LASTC)
        pltpu.async_copy(t_hbm.at[:, pl.ds(c0 * CHW, CHW)],
                         chunk_v.at[c0 & 1], csem)
        chunk_wait()
        c1 = jnp.minimum(c0 + 1, LASTC)
        pltpu.async_copy(t_hbm.at[:, pl.ds(c1 * CHW, CHW)],
                         chunk_v.at[(c0 + 1) & 1], csem)

        def group(g, carry):
            cur = carry
            rvec = rs_v[pl.ds(g * LANES, LANES)]
            bvec = bs_v[pl.ds(g * LANES, LANES)]
            for l in range(LANES):
                r = rvec[l]
                b = bvec[l]
                is_tail = r >= TAIL0
                cs = jnp.where(is_tail, cur, jnp.minimum(r >> CSH, LASTC))
                switch = cs != cur
                pbuf = cs & 1

                @pl.when(switch)
                def _(cs=cs, cur=cur):
                    chunk_wait()

                    @pl.when(cs != cur + 1)
                    def _():
                        pltpu.async_copy(t_hbm.at[:, pl.ds(cs * CHW, CHW)],
                                         chunk_v.at[cs & 1], csem)
                        chunk_wait()

                    cn = jnp.minimum(cs + 1, LASTC)
                    pltpu.async_copy(t_hbm.at[:, pl.ds(cn * CHW, CHW)],
                                     chunk_v.at[(cs + 1) & 1], csem)

                slot = (l % RING) * EMB
                if l >= RING:
                    row_wait()
                else:
                    @pl.when(g > 0)
                    def _():
                        row_wait()

                rel = jnp.clip(r - cs * CHW, 0, CHW - 1)
                p_vec = jnp.full((LANES,), pbuf, jnp.int32)
                rel_vec = jnp.full((LANES,), rel, jnp.int32)
                tbase = jnp.clip(r - TAIL0, 0, TAILN - 1) * EMB
                tail_flag = jnp.full((LANES,), is_tail, jnp.bool_)
                for k in range(EMB // LANES):
                    dk = lane + k * LANES
                    main_vals = plsc.load_gather(
                        chunk_v, [p_vec, dk, rel_vec])
                    tail_vals = plsc.load_gather(tail_v, [tbase + dk])
                    vals = jnp.where(tail_flag, tail_vals, main_vals)
                    ring_v[pl.ds(slot + k * LANES, LANES)] = vals

                pltpu.async_copy(ring_v.at[pl.ds(slot, EMB)],
                                 stage_hbm.at[pl.ds(b * EMB, EMB)], rsem)
                cur = cs
            return cur

        lax.fori_loop(0, HPW // LANES, group, c0)
        for _ in range(RING):
            row_wait()
        chunk_wait()

    one_table(us_hbm, uo_hbm, u_t_hbm, u_tail_hbm, su_hbm)
    one_table(vs_hbm, vo_hbm, v_t_hbm, v_tail_hbm, sv_hbm)


@functools.partial(
    pl.kernel,
    mesh=_mesh,
    compiler_params=_params,
    out_type=jax.ShapeDtypeStruct((BATCH,), jnp.float32),
    scratch_types=[
        pltpu.VMEM((HPW * EMB,), jnp.float32),
        pltpu.VMEM((HPW * EMB,), jnp.float32),
        pltpu.VMEM((HPW,), jnp.float32),
        pltpu.SemaphoreType.DMA,
    ],
)
def _dot_sc(su_hbm, sv_hbm, out_hbm, uf_v, vf_v, out_v, sem):
    wid = lax.axis_index("s") * NC + lax.axis_index("c")
    base = wid * HPW
    cu = pltpu.async_copy(su_hbm.at[pl.ds(base * EMB, HPW * EMB)], uf_v, sem)
    cv = pltpu.async_copy(sv_hbm.at[pl.ds(base * EMB, HPW * EMB)], vf_v, sem)
    cu.wait()
    cv.wait()

    lane = lax.iota(jnp.int32, LANES)

    def blk(i, carry):
        res = jnp.zeros((LANES,), jnp.float32)
        for l in range(LANES):
            roff = (i * LANES + l) * EMB
            acc = jnp.zeros((LANES,), jnp.float32)
            for k in range(EMB // LANES):
                u = uf_v[pl.ds(roff + k * LANES, LANES)]
                v = vf_v[pl.ds(roff + k * LANES, LANES)]
                acc = acc + u * v
            s = jnp.sum(acc)
            res = jnp.where(lane == l, s, res)
        out_v[pl.ds(i * LANES, LANES)] = res
        return carry

    lax.fori_loop(0, HPW // LANES, blk, 0)
    pltpu.sync_copy(out_v, out_hbm.at[pl.ds(base, HPW)])


def kernel(u_idxs, v_idxs, u_weight, v_weight):
    ui = u_idxs.astype(jnp.int32)
    vi = v_idxs.astype(jnp.int32)
    uo = jnp.argsort(ui).astype(jnp.int32)
    vo = jnp.argsort(vi).astype(jnp.int32)
    us = ui[uo]
    vs = vi[vo]
    u_tail = u_weight[TAIL0:, :].reshape(-1)
    v_tail = v_weight[TAIL0:, :].reshape(-1)
    su, sv = _gather_sc(us, uo, vs, vo, u_weight.T, v_weight.T,
                        u_tail, v_tail)
    return _dot_sc(su, sv)


# speedup vs baseline: 1.7144x; 1.0497x over previous
import functools

import jax
import jax.numpy as jnp
from jax import lax
from jax.experimental import pallas as pl
from jax.experimental.pallas import tpu as pltpu
from jax.experimental.pallas import tpu_sc as plsc

VOCAB = 1000000
EMB = 64
BATCH = 16384

NC = 2
NS = 16
NW = NC * NS
HPW = BATCH // NW
LANES = 16

CHW = 512
CSH = 9
TAIL0 = (VOCAB // 128) * 128
TAILN = VOCAB - TAIL0
LASTC = TAIL0 // CHW - 1
RING = 8

_mesh = plsc.VectorSubcoreMesh(core_axis_name="c", subcore_axis_name="s")
_params = pltpu.CompilerParams(
    needs_layout_passes=False, use_tc_tiling_on_sc=True)


@functools.partial(
    pl.kernel,
    mesh=_mesh,
    compiler_params=_params,
    out_type=(jax.ShapeDtypeStruct((BATCH * EMB,), jnp.float32),
              jax.ShapeDtypeStruct((BATCH * EMB,), jnp.float32)),
    scratch_types=[
        pltpu.VMEM((2, EMB, CHW), jnp.float32),
        pltpu.VMEM((TAILN * EMB,), jnp.float32),
        pltpu.VMEM((RING * EMB,), jnp.float32),
        pltpu.VMEM((HPW + LANES,), jnp.int32),
        pltpu.VMEM((HPW,), jnp.int32),
        pltpu.SemaphoreType.DMA,
        pltpu.SemaphoreType.DMA,
    ],
)
def _gather_sc(us_hbm, uo_hbm, vs_hbm, vo_hbm, u_t_hbm, v_t_hbm,
               u_tail_hbm, v_tail_hbm, su_hbm, sv_hbm,
               chunk_v, tail_v, ring_v, rs_v, bs_v, csem, rsem):
    wid = lax.axis_index("s") * NC + lax.axis_index("c")
    j0 = wid * HPW
    lane = lax.iota(jnp.int32, LANES)

    def chunk_wait():
        pltpu.make_async_copy(
            u_t_hbm.at[:, pl.ds(0, CHW)], chunk_v.at[0], csem).wait()

    def row_wait():
        pltpu.make_async_copy(
            su_hbm.at[pl.ds(0, EMB)], ring_v.at[pl.ds(0, EMB)], rsem).wait()

    def one_table(s_hbm, o_hbm, t_hbm, tail_hbm, stage_hbm):
        pltpu.sync_copy(s_hbm.at[pl.ds(j0, HPW)], rs_v.at[pl.ds(0, HPW)])
        rs_v[pl.ds(HPW, LANES)] = jnp.full((LANES,), VOCAB, jnp.int32)
        pltpu.sync_copy(o_hbm.at[pl.ds(j0, HPW)], bs_v)
        pltpu.sync_copy(tail_hbm, tail_v)

        c0 = jnp.minimum(rs_v[pl.ds(0, LANES)][0] >> CSH, LASTC)
        pltpu.async_copy(t_hbm.at[:, pl.ds(c0 * CHW, CHW)],
                         chunk_v.at[c0 & 1], csem)
        chunk_wait()
        c1 = jnp.minimum(c0 + 1, LASTC)
        pltpu.async_copy(t_hbm.at[:, pl.ds(c1 * CHW, CHW)],
                         chunk_v.at[(c0 + 1) & 1], csem)

        def group(g, carry):
            cur = carry
            rvec = rs_v[pl.ds(g * LANES, LANES)]
            bvec = bs_v[pl.ds(g * LANES, LANES)]
            for l in range(LANES):
                r = rvec[l]
                b = bvec[l]
                is_tail = r >= TAIL0
                cs = jnp.where(is_tail, cur, jnp.minimum(r >> CSH, LASTC))
                switch = cs != cur
                pbuf = cs & 1

                @pl.when(switch)
                def _(cs=cs, cur=cur):
                    chunk_wait()

                    @pl.when(cs != cur + 1)
                    def _():
                        pltpu.async_copy(t_hbm.at[:, pl.ds(cs * CHW, CHW)],
                                         chunk_v.at[cs & 1], csem)
                        chunk_wait()

                    cn = jnp.minimum(cs + 1, LASTC)
                    pltpu.async_copy(t_hbm.at[:, pl.ds(cn * CHW, CHW)],
                                     chunk_v.at[(cs + 1) & 1], csem)

                slot = (l % RING) * EMB
                if l >= RING:
                    row_wait()
                else:
                    @pl.when(g > 0)
                    def _():
                        row_wait()

                rel = jnp.clip(r - cs * CHW, 0, CHW - 1)
                p_vec = jnp.full((LANES,), pbuf, jnp.int32)
                rel_vec = jnp.full((LANES,), rel, jnp.int32)
                tbase = jnp.clip(r - TAIL0, 0, TAILN - 1) * EMB
                tail_flag = jnp.full((LANES,), is_tail, jnp.bool_)
                for k in range(EMB // LANES):
                    dk = lane + k * LANES
                    main_vals = plsc.load_gather(
                        chunk_v, [p_vec, dk, rel_vec])
                    tail_vals = plsc.load_gather(tail_v, [tbase + dk])
                    vals = jnp.where(tail_flag, tail_vals, main_vals)
                    ring_v[pl.ds(slot + k * LANES, LANES)] = vals

                pltpu.async_copy(ring_v.at[pl.ds(slot, EMB)],
                                 stage_hbm.at[pl.ds(b * EMB, EMB)], rsem)
                cur = cs
            return cur

        lax.fori_loop(0, HPW // LANES, group, c0)
        for _ in range(RING):
            row_wait()
        chunk_wait()

    one_table(us_hbm, uo_hbm, u_t_hbm, u_tail_hbm, su_hbm)
    one_table(vs_hbm, vo_hbm, v_t_hbm, v_tail_hbm, sv_hbm)


@functools.partial(
    pl.kernel,
    mesh=_mesh,
    compiler_params=_params,
    out_type=jax.ShapeDtypeStruct((BATCH,), jnp.float32),
    scratch_types=[
        pltpu.VMEM((HPW * EMB,), jnp.float32),
        pltpu.VMEM((HPW * EMB,), jnp.float32),
        pltpu.VMEM((HPW,), jnp.float32),
        pltpu.SemaphoreType.DMA,
    ],
)
def _dot_sc(su_hbm, sv_hbm, out_hbm, uf_v, vf_v, out_v, sem):
    wid = lax.axis_index("s") * NC + lax.axis_index("c")
    base = wid * HPW
    cu = pltpu.async_copy(su_hbm.at[pl.ds(base * EMB, HPW * EMB)], uf_v, sem)
    cv = pltpu.async_copy(sv_hbm.at[pl.ds(base * EMB, HPW * EMB)], vf_v, sem)
    cu.wait()
    cv.wait()

    lane = lax.iota(jnp.int32, LANES)

    def blk(i, carry):
        res = jnp.zeros((LANES,), jnp.float32)
        for l in range(LANES):
            roff = (i * LANES + l) * EMB
            acc = jnp.zeros((LANES,), jnp.float32)
            for k in range(EMB // LANES):
                u = uf_v[pl.ds(roff + k * LANES, LANES)]
                v = vf_v[pl.ds(roff + k * LANES, LANES)]
                acc = acc + u * v
            s = jnp.sum(acc)
            res = jnp.where(lane == l, s, res)
        out_v[pl.ds(i * LANES, LANES)] = res
        return carry

    lax.fori_loop(0, HPW // LANES, blk, 0)
    pltpu.sync_copy(out_v, out_hbm.at[pl.ds(base, HPW)])


def kernel(u_idxs, v_idxs, u_weight, v_weight):
    ui = u_idxs.astype(jnp.int32)
    vi = v_idxs.astype(jnp.int32)
    iot = lax.iota(jnp.int32, BATCH)
    us, uo = lax.sort_key_val(ui, iot)
    vs, vo = lax.sort_key_val(vi, iot)
    u_tail = u_weight[TAIL0:, :].reshape(-1)
    v_tail = v_weight[TAIL0:, :].reshape(-1)
    su, sv = _gather_sc(us, uo, vs, vo, u_weight.T, v_weight.T,
                        u_tail, v_tail)
    return _dot_sc(su, sv)
